# baseline (speedup 1.0000x reference)
import functools

import jax
import jax.numpy as jnp
from jax import lax
from jax.experimental import pallas as pl
from jax.experimental.pallas import tpu as pltpu
from jax.experimental.pallas import tpu_sc as plsc

NUM_EMB = 100000
DIM = 64
E = 16384

NUM_CORES = 2
NUM_SUBCORES = 16
LANES = 16
EPC = E // NUM_CORES
FPS = DIM // NUM_SUBCORES
AROWS = EPC // 128
SROWS = AROWS // NUM_SUBCORES


def _sc_body(eidx_hbm, tabT_hbm, out_hbm,
             rowbuf, idxa_v, idxb_v, acc_v, idxid_v, red_v, spsum):
    core = lax.axis_index("c")
    sub = lax.axis_index("s")
    ebase = core * EPC

    lanes = lax.iota(jnp.int32, LANES)

    pltpu.sync_copy(eidx_hbm.at[pl.ds(ebase, EPC)], idxa_v)
    pltpu.sync_copy(eidx_hbm.at[pl.ds(E + ebase, EPC)], idxb_v)

    for j in range(AROWS // LANES):
        idxid_v[pl.ds(j * LANES, LANES)] = j * LANES + lanes

    @pl.when(sub == 0)
    def _():
        zero = jnp.zeros((LANES,), jnp.float32)

        @pl.loop(0, AROWS)
        def _(r):
            for k in range(128 // LANES):
                acc_v[r, pl.ds(k * LANES, LANES)] = zero

        pltpu.sync_copy(acc_v, spsum)

    plsc.subcore_barrier()

    for cl in range(FPS):
        feat = sub * FPS + cl
        pltpu.sync_copy(tabT_hbm.at[feat], rowbuf)

        @pl.loop(0, AROWS)
        def _(r):
            base = r * 128
            for k in range(128 // LANES):
                off = base + k * LANES
                ia = idxa_v[pl.ds(off, LANES)]
                ib = idxb_v[pl.ds(off, LANES)]
                p = (plsc.load_gather(rowbuf, [ia])
                     * plsc.load_gather(rowbuf, [ib]))
                dst = pl.ds(k * LANES, LANES)
                if cl == 0:
                    acc_v[r, dst] = p
                else:
                    acc_v[r, dst] = acc_v[r, dst] + p

    pltpu.sync_copy(acc_v, spsum.at[idxid_v], add=True)
    plsc.subcore_barrier()

    pltpu.sync_copy(spsum.at[pl.ds(sub * SROWS, SROWS)], red_v)
    for r in range(SROWS):
        for k in range(128 // LANES):
            s = pl.ds(k * LANES, LANES)
            red_v[r, s] = 1.0 / (1.0 + jnp.exp(-red_v[r, s]))
    pltpu.sync_copy(red_v, out_hbm.at[core, sub])


def kernel(edges, emb_table):
    eidx = edges.astype(jnp.int32).reshape(2 * E)
    tabT = emb_table.T
    mesh = plsc.VectorSubcoreMesh(core_axis_name="c", subcore_axis_name="s")
    sc = functools.partial(
        pl.kernel,
        mesh=mesh,
        compiler_params=pltpu.CompilerParams(needs_layout_passes=False),
        out_type=jax.ShapeDtypeStruct(
            (NUM_CORES, NUM_SUBCORES, SROWS, 128), jnp.float32),
        scratch_types=[
            pltpu.VMEM((NUM_EMB,), jnp.float32),
            pltpu.VMEM((EPC,), jnp.int32),
            pltpu.VMEM((EPC,), jnp.int32),
            pltpu.VMEM((AROWS, 128), jnp.float32),
            pltpu.VMEM((AROWS,), jnp.int32),
            pltpu.VMEM((SROWS, 128), jnp.float32),
            pltpu.VMEM_SHARED((AROWS, 128), jnp.float32),
        ],
    )(_sc_body)
    return sc(eidx, tabT).reshape(E)
